# baseline (speedup 1.0000x reference)
import functools

import jax
import jax.numpy as jnp
from jax.experimental import pallas as pl
from jax.experimental.pallas import tpu as pltpu


def _coeffs(s_ref, ss_ref, g_ref, be_ref, n):
    mean = s_ref[:] * (1.0 / n)
    var = ss_ref[:] * (1.0 / n) - mean * mean
    scale = g_ref[:] / jnp.sqrt(var + 1e-5)
    shift = be_ref[:] - mean * scale
    return scale, shift


def _interp_body(*refs, k, n_src, affine, n_prev, win):
    if win:
        st_ref, *refs = refs
    bq_ref, q_ref, bs_ref, pt_ref, x_ref, *rest = refs
    if affine:
        s_ref, ss_ref, g_ref, be_ref, o_ref = rest
    else:
        (o_ref,) = rest
    if win:
        st = pl.multiple_of(st_ref[pl.program_id(0)], 256)
        pt = pt_ref[:, pl.ds(st, win)]
        bs = bs_ref[:, pl.ds(st, win)]
        x = x_ref[pl.ds(st, win), :]
        n_src = win
    else:
        pt = pt_ref[:]
        bs = bs_ref[:]
        x = x_ref[:]
    q = q_ref[:]
    q0, q1, q2 = q[:, 0:1], q[:, 1:2], q[:, 2:3]
    p0 = pt[0:1, :]
    p1 = pt[1:2, :]
    p2 = pt[2:3, :]
    qn = q0 * q0 + q1 * q1 + q2 * q2
    pn = p0 * p0 + p1 * p1 + p2 * p2
    cross = jnp.dot(q.astype(jnp.bfloat16), pt.astype(jnp.bfloat16),
                    preferred_element_type=jnp.float32)
    d2 = (qn - 2.0 * cross) + pn
    bmask = bq_ref[:] != bs
    d2 = jnp.where(bmask, 1e10, d2)

    iota = jax.lax.broadcasted_iota(jnp.int32, d2.shape, 1)
    S = None
    wsum = None
    for j in range(k):
        m = jnp.min(d2, axis=1, keepdims=True)
        am = jnp.min(jnp.where(d2 == m, iota, n_src), axis=1, keepdims=True)
        sel = iota == am
        if k == 1:
            S = sel.astype(jnp.float32)
        else:
            w = 1.0 / jnp.clip(m, 1e-16, None)
            contrib = jnp.where(sel, w, 0.0)
            S = contrib if S is None else S + contrib
            wsum = w if wsum is None else wsum + w
            if j + 1 < k:
                d2 = jnp.where(sel, 1e30, d2)

    out = jnp.dot(S, x, preferred_element_type=jnp.float32)
    if k > 1:
        out = out / wsum
    if affine:
        scale, shift = _coeffs(s_ref, ss_ref, g_ref, be_ref, n_prev)
        out = out * scale + shift
    o_ref[:] = out.astype(o_ref.dtype)


def _interp(x, pos_src_t, pos_q, bs_row, bq_col, k, stats=None, n_prev=None,
            bq=256, starts=None, win=None):
    n_src = pos_src_t.shape[1]
    n_q = pos_q.shape[0]
    c = x.shape[1]
    bq = min(bq, n_q)
    grid = (n_q // bq,)
    if win is None or win >= n_src:
        starts, win = None, None
    nsp = 1 if starts is not None else 0
    ix = lambda i: (i, 0)
    fix = lambda i: (0, 0)
    if nsp:
        ix = lambda i, _s: (i, 0)
        fix = lambda i, _s: (0, 0)
    in_specs = [
        pl.BlockSpec((bq, 1), ix),
        pl.BlockSpec((bq, 3), ix),
        pl.BlockSpec((1, n_src), fix),
        pl.BlockSpec((3, n_src), fix),
        pl.BlockSpec((n_src, c), fix),
    ]
    args = [bq_col, pos_q, bs_row, pos_src_t, x]
    if stats is not None:
        cs = pl.BlockSpec((1, c), fix)
        in_specs += [cs, cs, cs, cs]
        s, ss, g, be = stats
        args += [s, ss, g.reshape(1, c), be.reshape(1, c)]
    body = functools.partial(_interp_body, k=k, n_src=n_src,
                             affine=stats is not None,
                             n_prev=float(n_prev) if n_prev else 1.0,
                             win=win)
    out_spec = pl.BlockSpec((bq, c), ix)
    out_shape = jax.ShapeDtypeStruct((n_q, c), jnp.bfloat16)
    if nsp:
        gs = pltpu.PrefetchScalarGridSpec(
            num_scalar_prefetch=1, grid=grid,
            in_specs=in_specs, out_specs=out_spec)
        return pl.pallas_call(body, grid_spec=gs,
                              out_shape=out_shape)(starts, *args)
    return pl.pallas_call(
        body,
        grid=grid,
        in_specs=in_specs,
        out_specs=out_spec,
        out_shape=out_shape,
    )(*args)


def _interp_lin_body(st_ref, bq_ref, q_ref, bs_ref, pt_ref, x_ref,
                     s_ref, ss_ref, g_ref, be_ref, w_ref, b_ref, xs_ref,
                     h_ref, so_ref, sso_ref, *, win, n_prev, kx):
    st = pl.multiple_of(st_ref[pl.program_id(0)], 256)
    pt = pt_ref[:, pl.ds(st, win)]
    bs = bs_ref[:, pl.ds(st, win)]
    x = x_ref[pl.ds(st, win), :]
    q = q_ref[:]
    q0, q1, q2 = q[:, 0:1], q[:, 1:2], q[:, 2:3]
    p0, p1, p2 = pt[0:1, :], pt[1:2, :], pt[2:3, :]
    qn = q0 * q0 + q1 * q1 + q2 * q2
    pn = p0 * p0 + p1 * p1 + p2 * p2
    cross = jnp.dot(q.astype(jnp.bfloat16), pt.astype(jnp.bfloat16),
                    preferred_element_type=jnp.float32)
    d2 = (qn - 2.0 * cross) + pn
    d2 = jnp.where(bq_ref[:] != bs, 1e10, d2)

    iota = jax.lax.broadcasted_iota(jnp.int32, d2.shape, 1)
    S = None
    wsum = None
    for j in range(3):
        m = jnp.min(d2, axis=1, keepdims=True)
        am = jnp.min(jnp.where(d2 == m, iota, win), axis=1, keepdims=True)
        sel = iota == am
        w = 1.0 / jnp.clip(m, 1e-16, None)
        contrib = jnp.where(sel, w, 0.0)
        S = contrib if S is None else S + contrib
        wsum = w if wsum is None else wsum + w
        if j < 2:
            d2 = jnp.where(sel, 1e30, d2)

    xi = jnp.dot(S, x, preferred_element_type=jnp.float32) / wsum
    scale, shift = _coeffs(s_ref, ss_ref, g_ref, be_ref, n_prev)
    xi = (xi * scale + shift).astype(jnp.bfloat16)
    h = jnp.dot(xi, w_ref[0:kx, :].astype(jnp.bfloat16),
                preferred_element_type=jnp.float32)
    h = h + jnp.dot(xs_ref[:], w_ref[kx:, :],
                    preferred_element_type=jnp.float32)
    h = jnp.maximum(h + b_ref[:], 0.0)
    h_ref[:] = h
    csum = jnp.sum(h, axis=0, keepdims=True)
    csq = jnp.sum(h * h, axis=0, keepdims=True)

    @pl.when(pl.program_id(0) == 0)
    def _init():
        so_ref[:] = csum
        sso_ref[:] = csq

    @pl.when(pl.program_id(0) > 0)
    def _acc():
        so_ref[:] += csum
        sso_ref[:] += csq


def _interp_lin(x, pos_src_t, pos_q, bs_row, bq_col, stats, n_prev,
                w, b_row, skip, starts, win, bq=256):
    n_src = pos_src_t.shape[1]
    n_q = pos_q.shape[0]
    c = x.shape[1]
    fout = w.shape[1]
    ks = skip.shape[1]
    grid = (n_q // bq,)
    ix = lambda i, _s: (i, 0)
    fix = lambda i, _s: (0, 0)
    cs = pl.BlockSpec((1, c), fix)
    s, ss, g, be = stats
    gs = pltpu.PrefetchScalarGridSpec(
        num_scalar_prefetch=1, grid=grid,
        in_specs=[
            pl.BlockSpec((bq, 1), ix),
            pl.BlockSpec((bq, 3), ix),
            pl.BlockSpec((1, n_src), fix),
            pl.BlockSpec((3, n_src), fix),
            pl.BlockSpec((n_src, c), fix),
            cs, cs, cs, cs,
            pl.BlockSpec(w.shape, fix),
            pl.BlockSpec((1, fout), fix),
            pl.BlockSpec((bq, ks), ix),
        ],
        out_specs=[
            pl.BlockSpec((bq, fout), ix),
            pl.BlockSpec((1, fout), fix),
            pl.BlockSpec((1, fout), fix),
        ])
    return pl.pallas_call(
        functools.partial(_interp_lin_body, win=win, n_prev=float(n_prev),
                          kx=c),
        grid_spec=gs,
        out_shape=[
            jax.ShapeDtypeStruct((n_q, fout), jnp.float32),
            jax.ShapeDtypeStruct((1, fout), jnp.float32),
            jax.ShapeDtypeStruct((1, fout), jnp.float32),
        ],
    )(starts, bq_col, pos_q, bs_row, pos_src_t, x,
      s, ss, g.reshape(1, c), be.reshape(1, c), w, b_row, skip)


def _linear_body(x_ref, w_ref, b_ref, *rest, affine, skip, kx, n_prev):
    rest = list(rest)
    xs_ref = rest.pop(0) if skip else None
    if affine:
        s_ref, ss_ref, g_ref, be_ref = rest[:4]
        rest = rest[4:]
    h_ref, so_ref, sso_ref = rest
    x = x_ref[:]
    if affine:
        scale, shift = _coeffs(s_ref, ss_ref, g_ref, be_ref, n_prev)
        x = x * scale + shift
    w_head = w_ref[0:kx, :]
    if x.dtype == jnp.bfloat16:
        w_head = w_head.astype(jnp.bfloat16)
    h = jnp.dot(x, w_head, preferred_element_type=jnp.float32)
    if skip:
        h = h + jnp.dot(xs_ref[:], w_ref[kx:, :],
                        preferred_element_type=jnp.float32)
    h = h + b_ref[:]
    h = jnp.maximum(h, 0.0)
    h_ref[:] = h
    csum = jnp.sum(h, axis=0, keepdims=True)
    csq = jnp.sum(h * h, axis=0, keepdims=True)

    @pl.when(pl.program_id(0) == 0)
    def _init():
        so_ref[:] = csum
        sso_ref[:] = csq

    @pl.when(pl.program_id(0) > 0)
    def _acc():
        so_ref[:] += csum
        sso_ref[:] += csq


def _linear(x, w, b_row, skip=None, stats=None, n_prev=None, br=512):
    n, kx = x.shape
    fout = w.shape[1]
    br = min(br, n)
    grid = (n // br,)
    in_specs = [
        pl.BlockSpec((br, kx), lambda i: (i, 0)),
        pl.BlockSpec(w.shape, lambda i: (0, 0)),
        pl.BlockSpec((1, fout), lambda i: (0, 0)),
    ]
    args = [x, w, b_row]
    if skip is not None:
        ks = skip.shape[1]
        in_specs.append(pl.BlockSpec((br, ks), lambda i: (i, 0)))
        args.append(skip)
    if stats is not None:
        cs = pl.BlockSpec((1, kx), lambda i: (0, 0))
        in_specs += [cs, cs, cs, cs]
        s, ss, g, be = stats
        args += [s, ss, g.reshape(1, kx), be.reshape(1, kx)]
    return pl.pallas_call(
        functools.partial(_linear_body, affine=stats is not None,
                          skip=skip is not None, kx=kx,
                          n_prev=float(n_prev) if n_prev else 1.0),
        grid=grid,
        in_specs=in_specs,
        out_specs=[
            pl.BlockSpec((br, fout), lambda i: (i, 0)),
            pl.BlockSpec((1, fout), lambda i: (0, 0)),
            pl.BlockSpec((1, fout), lambda i: (0, 0)),
        ],
        out_shape=[
            jax.ShapeDtypeStruct((n, fout), jnp.float32),
            jax.ShapeDtypeStruct((1, fout), jnp.float32),
            jax.ShapeDtypeStruct((1, fout), jnp.float32),
        ],
    )(*args)


def _bn_apply_body(h_ref, s_ref, ss_ref, g_ref, be_ref, o_ref, *, n_prev):
    scale, shift = _coeffs(s_ref, ss_ref, g_ref, be_ref, n_prev)
    o_ref[:] = h_ref[:] * scale + shift


def _bn_apply(h, stats, n_prev, br=2048):
    n, f = h.shape
    s, ss, g, be = stats
    br = min(br, n)
    grid = (n // br,)
    cs = pl.BlockSpec((1, f), lambda i: (0, 0))
    return pl.pallas_call(
        functools.partial(_bn_apply_body, n_prev=float(n_prev)),
        grid=grid,
        in_specs=[pl.BlockSpec((br, f), lambda i: (i, 0)), cs, cs, cs, cs],
        out_specs=pl.BlockSpec((br, f), lambda i: (i, 0)),
        out_shape=jax.ShapeDtypeStruct((n, f), jnp.float32),
    )(h, s, ss, g.reshape(1, f), be.reshape(1, f))


def kernel(x0, x1, x2, x3, p0, p1, p2, p3, b0, b1, b2, b3,
           l3w1, l3b1, l3g1, l3be1, l3w2, l3b2, l3g2, l3be2,
           l2w1, l2b1, l2g1, l2be1, l2w2, l2b2, l2g2, l2be2,
           l1w1, l1b1, l1g1, l1be1, l1w2, l1b2, l1g2, l1be2):
    n0, n1, n2 = x0.shape[0], x1.shape[0], x2.shape[0]
    b0c = b0.reshape(-1, 1)
    b1c = b1.reshape(-1, 1)
    b2c = b2.reshape(-1, 1)
    b1r = b1.reshape(1, -1)
    b2r = b2.reshape(1, -1)
    b3r = b3.reshape(1, -1)

    h = _interp(x3, p3.T, p2, b3r, b2c, k=1)
    h, s, ss = _linear(h, l3w1, l3b1.reshape(1, -1), skip=x2)
    h, s, ss = _linear(h, l3w2, l3b2.reshape(1, -1),
                       stats=(s, ss, l3g1, l3be1), n_prev=n2)

    win2 = 768
    nb2 = n1 // 256
    fq2 = jnp.arange(nb2) * 256
    seg2 = jnp.searchsorted(b2, b1[fq2], side="left").astype(jnp.int32)
    starts2 = jnp.maximum(
        jnp.minimum((seg2 // 256) * 256, jnp.int32(n2 - win2)), 0)
    h = _interp(h, p2.T, p1, b2r, b1c, k=3,
                stats=(s, ss, l3g2, l3be2), n_prev=n2,
                starts=starts2, win=win2)
    h, s, ss = _linear(h, l2w1, l2b1.reshape(1, -1), skip=x1)
    h, s, ss = _linear(h, l2w2, l2b2.reshape(1, -1),
                       stats=(s, ss, l2g1, l2be1), n_prev=n1)

    win = 2560
    nb = n0 // 256
    fq = jnp.arange(nb) * 256
    seg_start = jnp.searchsorted(b1, b0[fq], side="left").astype(jnp.int32)
    starts = jnp.minimum((seg_start // 256) * 256,
                         jnp.int32(x1.shape[0] - win))
    starts = jnp.maximum(starts, 0)
    h, s, ss = _interp_lin(h, p1.T, p0, b1r, b0c,
                           stats=(s, ss, l2g2, l2be2), n_prev=n1,
                           w=l1w1, b_row=l1b1.reshape(1, -1), skip=x0,
                           starts=starts, win=win)
    h, s, ss = _linear(h, l1w2, l1b2.reshape(1, -1),
                       stats=(s, ss, l1g1, l1be1), n_prev=n0)

    return _bn_apply(h, (s, ss, l1g2, l1be2), n0)


# speedup vs baseline: 1.0177x; 1.0177x over previous
import functools

import jax
import jax.numpy as jnp
from jax.experimental import pallas as pl
from jax.experimental.pallas import tpu as pltpu


def _coeffs(s_ref, ss_ref, g_ref, be_ref, n):
    mean = s_ref[:] * (1.0 / n)
    var = ss_ref[:] * (1.0 / n) - mean * mean
    scale = g_ref[:] / jnp.sqrt(var + 1e-5)
    shift = be_ref[:] - mean * scale
    return scale, shift


def _interp_body(*refs, k, n_src, affine, n_prev, win):
    if win:
        st_ref, *refs = refs
    bq_ref, q_ref, bs_ref, pt_ref, x_ref, *rest = refs
    if affine:
        s_ref, ss_ref, g_ref, be_ref, o_ref = rest
    else:
        (o_ref,) = rest
    if win:
        st = pl.multiple_of(st_ref[pl.program_id(0)], 128)
        pt = pt_ref[:, pl.ds(st, win)]
        bs = bs_ref[:, pl.ds(st, win)]
        x = x_ref[pl.ds(st, win), :]
        n_src = win
    else:
        pt = pt_ref[:]
        bs = bs_ref[:]
        x = x_ref[:]
    q = q_ref[:]
    q0, q1, q2 = q[:, 0:1], q[:, 1:2], q[:, 2:3]
    p0 = pt[0:1, :]
    p1 = pt[1:2, :]
    p2 = pt[2:3, :]
    qn = q0 * q0 + q1 * q1 + q2 * q2
    pn = p0 * p0 + p1 * p1 + p2 * p2
    cross = jnp.dot(q.astype(jnp.bfloat16), pt.astype(jnp.bfloat16),
                    preferred_element_type=jnp.float32)
    d2 = (qn - 2.0 * cross) + pn
    bmask = bq_ref[:] != bs
    d2 = jnp.where(bmask, 1e10, d2)

    iota = jax.lax.broadcasted_iota(jnp.int32, d2.shape, 1)
    S = None
    wsum = None
    for j in range(k):
        m = jnp.min(d2, axis=1, keepdims=True)
        am = jnp.min(jnp.where(d2 == m, iota, n_src), axis=1, keepdims=True)
        sel = iota == am
        if k == 1:
            S = sel.astype(jnp.float32)
        else:
            w = 1.0 / jnp.clip(m, 1e-16, None)
            contrib = jnp.where(sel, w, 0.0)
            S = contrib if S is None else S + contrib
            wsum = w if wsum is None else wsum + w
            if j + 1 < k:
                d2 = jnp.where(sel, 1e30, d2)

    out = jnp.dot(S, x, preferred_element_type=jnp.float32)
    if k > 1:
        out = out / wsum
    if affine:
        scale, shift = _coeffs(s_ref, ss_ref, g_ref, be_ref, n_prev)
        out = out * scale + shift
    o_ref[:] = out.astype(o_ref.dtype)


def _interp(x, pos_src_t, pos_q, bs_row, bq_col, k, stats=None, n_prev=None,
            bq=256, starts=None, win=None):
    n_src = pos_src_t.shape[1]
    n_q = pos_q.shape[0]
    c = x.shape[1]
    bq = min(bq, n_q)
    grid = (n_q // bq,)
    if win is None or win >= n_src:
        starts, win = None, None
    nsp = 1 if starts is not None else 0
    ix = lambda i: (i, 0)
    fix = lambda i: (0, 0)
    if nsp:
        ix = lambda i, _s: (i, 0)
        fix = lambda i, _s: (0, 0)
    in_specs = [
        pl.BlockSpec((bq, 1), ix),
        pl.BlockSpec((bq, 3), ix),
        pl.BlockSpec((1, n_src), fix),
        pl.BlockSpec((3, n_src), fix),
        pl.BlockSpec((n_src, c), fix),
    ]
    args = [bq_col, pos_q, bs_row, pos_src_t, x]
    if stats is not None:
        cs = pl.BlockSpec((1, c), fix)
        in_specs += [cs, cs, cs, cs]
        s, ss, g, be = stats
        args += [s, ss, g.reshape(1, c), be.reshape(1, c)]
    body = functools.partial(_interp_body, k=k, n_src=n_src,
                             affine=stats is not None,
                             n_prev=float(n_prev) if n_prev else 1.0,
                             win=win)
    out_spec = pl.BlockSpec((bq, c), ix)
    out_shape = jax.ShapeDtypeStruct((n_q, c), jnp.bfloat16)
    if nsp:
        gs = pltpu.PrefetchScalarGridSpec(
            num_scalar_prefetch=1, grid=grid,
            in_specs=in_specs, out_specs=out_spec)
        return pl.pallas_call(body, grid_spec=gs,
                              out_shape=out_shape)(starts, *args)
    return pl.pallas_call(
        body,
        grid=grid,
        in_specs=in_specs,
        out_specs=out_spec,
        out_shape=out_shape,
    )(*args)


def _interp_lin_body(st_ref, bq_ref, q_ref, bs_ref, pt_ref, x_ref,
                     s_ref, ss_ref, g_ref, be_ref, w_ref, b_ref, xs_ref,
                     h_ref, so_ref, sso_ref, *, win, n_prev, kx):
    st = pl.multiple_of(st_ref[pl.program_id(0)], 128)
    pt = pt_ref[:, pl.ds(st, win)]
    bs = bs_ref[:, pl.ds(st, win)]
    x = x_ref[pl.ds(st, win), :]
    q = q_ref[:]
    q0, q1, q2 = q[:, 0:1], q[:, 1:2], q[:, 2:3]
    p0, p1, p2 = pt[0:1, :], pt[1:2, :], pt[2:3, :]
    qn = q0 * q0 + q1 * q1 + q2 * q2
    pn = p0 * p0 + p1 * p1 + p2 * p2
    cross = jnp.dot(q.astype(jnp.bfloat16), pt.astype(jnp.bfloat16),
                    preferred_element_type=jnp.float32)
    d2 = (qn - 2.0 * cross) + pn
    d2 = jnp.where(bq_ref[:] != bs, 1e10, d2)

    iota = jax.lax.broadcasted_iota(jnp.int32, d2.shape, 1)
    S = None
    wsum = None
    for j in range(3):
        m = jnp.min(d2, axis=1, keepdims=True)
        am = jnp.min(jnp.where(d2 == m, iota, win), axis=1, keepdims=True)
        sel = iota == am
        w = 1.0 / jnp.clip(m, 1e-16, None)
        contrib = jnp.where(sel, w, 0.0)
        S = contrib if S is None else S + contrib
        wsum = w if wsum is None else wsum + w
        if j < 2:
            d2 = jnp.where(sel, 1e30, d2)

    xi = jnp.dot(S, x, preferred_element_type=jnp.float32) / wsum
    scale, shift = _coeffs(s_ref, ss_ref, g_ref, be_ref, n_prev)
    xi = (xi * scale + shift).astype(jnp.bfloat16)
    h = jnp.dot(xi, w_ref[0:kx, :].astype(jnp.bfloat16),
                preferred_element_type=jnp.float32)
    h = h + jnp.dot(xs_ref[:], w_ref[kx:, :],
                    preferred_element_type=jnp.float32)
    h = jnp.maximum(h + b_ref[:], 0.0)
    h_ref[:] = h
    csum = jnp.sum(h, axis=0, keepdims=True)
    csq = jnp.sum(h * h, axis=0, keepdims=True)

    @pl.when(pl.program_id(0) == 0)
    def _init():
        so_ref[:] = csum
        sso_ref[:] = csq

    @pl.when(pl.program_id(0) > 0)
    def _acc():
        so_ref[:] += csum
        sso_ref[:] += csq


def _interp_lin(x, pos_src_t, pos_q, bs_row, bq_col, stats, n_prev,
                w, b_row, skip, starts, win, bq=256):
    n_src = pos_src_t.shape[1]
    n_q = pos_q.shape[0]
    c = x.shape[1]
    fout = w.shape[1]
    ks = skip.shape[1]
    grid = (n_q // bq,)
    ix = lambda i, _s: (i, 0)
    fix = lambda i, _s: (0, 0)
    cs = pl.BlockSpec((1, c), fix)
    s, ss, g, be = stats
    gs = pltpu.PrefetchScalarGridSpec(
        num_scalar_prefetch=1, grid=grid,
        in_specs=[
            pl.BlockSpec((bq, 1), ix),
            pl.BlockSpec((bq, 3), ix),
            pl.BlockSpec((1, n_src), fix),
            pl.BlockSpec((3, n_src), fix),
            pl.BlockSpec((n_src, c), fix),
            cs, cs, cs, cs,
            pl.BlockSpec(w.shape, fix),
            pl.BlockSpec((1, fout), fix),
            pl.BlockSpec((bq, ks), ix),
        ],
        out_specs=[
            pl.BlockSpec((bq, fout), ix),
            pl.BlockSpec((1, fout), fix),
            pl.BlockSpec((1, fout), fix),
        ])
    return pl.pallas_call(
        functools.partial(_interp_lin_body, win=win, n_prev=float(n_prev),
                          kx=c),
        grid_spec=gs,
        out_shape=[
            jax.ShapeDtypeStruct((n_q, fout), jnp.float32),
            jax.ShapeDtypeStruct((1, fout), jnp.float32),
            jax.ShapeDtypeStruct((1, fout), jnp.float32),
        ],
    )(starts, bq_col, pos_q, bs_row, pos_src_t, x,
      s, ss, g.reshape(1, c), be.reshape(1, c), w, b_row, skip)


def _linear_body(x_ref, w_ref, b_ref, *rest, affine, skip, kx, n_prev):
    rest = list(rest)
    xs_ref = rest.pop(0) if skip else None
    if affine:
        s_ref, ss_ref, g_ref, be_ref = rest[:4]
        rest = rest[4:]
    h_ref, so_ref, sso_ref = rest
    x = x_ref[:]
    if affine:
        scale, shift = _coeffs(s_ref, ss_ref, g_ref, be_ref, n_prev)
        x = x * scale + shift
    w_head = w_ref[0:kx, :]
    if x.dtype == jnp.bfloat16:
        w_head = w_head.astype(jnp.bfloat16)
    h = jnp.dot(x, w_head, preferred_element_type=jnp.float32)
    if skip:
        h = h + jnp.dot(xs_ref[:], w_ref[kx:, :],
                        preferred_element_type=jnp.float32)
    h = h + b_ref[:]
    h = jnp.maximum(h, 0.0)
    h_ref[:] = h
    csum = jnp.sum(h, axis=0, keepdims=True)
    csq = jnp.sum(h * h, axis=0, keepdims=True)

    @pl.when(pl.program_id(0) == 0)
    def _init():
        so_ref[:] = csum
        sso_ref[:] = csq

    @pl.when(pl.program_id(0) > 0)
    def _acc():
        so_ref[:] += csum
        sso_ref[:] += csq


def _linear(x, w, b_row, skip=None, stats=None, n_prev=None, br=512):
    n, kx = x.shape
    fout = w.shape[1]
    br = min(br, n)
    grid = (n // br,)
    in_specs = [
        pl.BlockSpec((br, kx), lambda i: (i, 0)),
        pl.BlockSpec(w.shape, lambda i: (0, 0)),
        pl.BlockSpec((1, fout), lambda i: (0, 0)),
    ]
    args = [x, w, b_row]
    if skip is not None:
        ks = skip.shape[1]
        in_specs.append(pl.BlockSpec((br, ks), lambda i: (i, 0)))
        args.append(skip)
    if stats is not None:
        cs = pl.BlockSpec((1, kx), lambda i: (0, 0))
        in_specs += [cs, cs, cs, cs]
        s, ss, g, be = stats
        args += [s, ss, g.reshape(1, kx), be.reshape(1, kx)]
    return pl.pallas_call(
        functools.partial(_linear_body, affine=stats is not None,
                          skip=skip is not None, kx=kx,
                          n_prev=float(n_prev) if n_prev else 1.0),
        grid=grid,
        in_specs=in_specs,
        out_specs=[
            pl.BlockSpec((br, fout), lambda i: (i, 0)),
            pl.BlockSpec((1, fout), lambda i: (0, 0)),
            pl.BlockSpec((1, fout), lambda i: (0, 0)),
        ],
        out_shape=[
            jax.ShapeDtypeStruct((n, fout), jnp.float32),
            jax.ShapeDtypeStruct((1, fout), jnp.float32),
            jax.ShapeDtypeStruct((1, fout), jnp.float32),
        ],
    )(*args)


def _bn_apply_body(h_ref, s_ref, ss_ref, g_ref, be_ref, o_ref, *, n_prev):
    scale, shift = _coeffs(s_ref, ss_ref, g_ref, be_ref, n_prev)
    o_ref[:] = h_ref[:] * scale + shift


def _bn_apply(h, stats, n_prev, br=2048):
    n, f = h.shape
    s, ss, g, be = stats
    br = min(br, n)
    grid = (n // br,)
    cs = pl.BlockSpec((1, f), lambda i: (0, 0))
    return pl.pallas_call(
        functools.partial(_bn_apply_body, n_prev=float(n_prev)),
        grid=grid,
        in_specs=[pl.BlockSpec((br, f), lambda i: (i, 0)), cs, cs, cs, cs],
        out_specs=pl.BlockSpec((br, f), lambda i: (i, 0)),
        out_shape=jax.ShapeDtypeStruct((n, f), jnp.float32),
    )(h, s, ss, g.reshape(1, f), be.reshape(1, f))


def kernel(x0, x1, x2, x3, p0, p1, p2, p3, b0, b1, b2, b3,
           l3w1, l3b1, l3g1, l3be1, l3w2, l3b2, l3g2, l3be2,
           l2w1, l2b1, l2g1, l2be1, l2w2, l2b2, l2g2, l2be2,
           l1w1, l1b1, l1g1, l1be1, l1w2, l1b2, l1g2, l1be2):
    n0, n1, n2 = x0.shape[0], x1.shape[0], x2.shape[0]
    b0c = b0.reshape(-1, 1)
    b1c = b1.reshape(-1, 1)
    b2c = b2.reshape(-1, 1)
    b1r = b1.reshape(1, -1)
    b2r = b2.reshape(1, -1)
    b3r = b3.reshape(1, -1)

    h = _interp(x3, p3.T, p2, b3r, b2c, k=1)
    h, s, ss = _linear(h, l3w1, l3b1.reshape(1, -1), skip=x2)
    h, s, ss = _linear(h, l3w2, l3b2.reshape(1, -1),
                       stats=(s, ss, l3g1, l3be1), n_prev=n2)

    h = _interp(h, p2.T, p1, b2r, b1c, k=3,
                stats=(s, ss, l3g2, l3be2), n_prev=n2)
    h, s, ss = _linear(h, l2w1, l2b1.reshape(1, -1), skip=x1)
    h, s, ss = _linear(h, l2w2, l2b2.reshape(1, -1),
                       stats=(s, ss, l2g1, l2be1), n_prev=n1)

    win = 2560
    nb = n0 // 256
    fq = jnp.arange(nb) * 256
    seg_start = jnp.searchsorted(b1, b0[fq], side="left").astype(jnp.int32)
    starts = jnp.minimum((seg_start // 128) * 128,
                         jnp.int32(x1.shape[0] - win))
    starts = jnp.maximum(starts, 0)
    h, s, ss = _interp_lin(h, p1.T, p0, b1r, b0c,
                           stats=(s, ss, l2g2, l2be2), n_prev=n1,
                           w=l1w1, b_row=l1b1.reshape(1, -1), skip=x0,
                           starts=starts, win=win)
    h, s, ss = _linear(h, l1w2, l1b2.reshape(1, -1),
                       stats=(s, ss, l1g1, l1be1), n_prev=n0)

    return _bn_apply(h, (s, ss, l1g2, l1be2), n0)


# speedup vs baseline: 1.0622x; 1.0437x over previous
import functools

import jax
import jax.numpy as jnp
from jax.experimental import pallas as pl
from jax.experimental.pallas import tpu as pltpu


def _coeffs(s_ref, ss_ref, g_ref, be_ref, n):
    mean = s_ref[:] * (1.0 / n)
    var = ss_ref[:] * (1.0 / n) - mean * mean
    scale = g_ref[:] / jnp.sqrt(var + 1e-5)
    shift = be_ref[:] - mean * scale
    return scale, shift


def _interp_body(*refs, k, n_src, affine, n_prev, win):
    if win:
        st_ref, *refs = refs
    bq_ref, q_ref, bs_ref, pt_ref, x_ref, *rest = refs
    if affine:
        s_ref, ss_ref, g_ref, be_ref, o_ref = rest
    else:
        (o_ref,) = rest
    if win:
        st = pl.multiple_of(st_ref[pl.program_id(0)], 128)
        pt = pt_ref[:, pl.ds(st, win)]
        bs = bs_ref[:, pl.ds(st, win)]
        x = x_ref[pl.ds(st, win), :]
        n_src = win
    else:
        pt = pt_ref[:]
        bs = bs_ref[:]
        x = x_ref[:]
    q = q_ref[:]
    q0, q1, q2 = q[:, 0:1], q[:, 1:2], q[:, 2:3]
    p0 = pt[0:1, :]
    p1 = pt[1:2, :]
    p2 = pt[2:3, :]
    qn = q0 * q0 + q1 * q1 + q2 * q2
    pn = p0 * p0 + p1 * p1 + p2 * p2
    cross = jnp.dot(q.astype(jnp.bfloat16), pt.astype(jnp.bfloat16),
                    preferred_element_type=jnp.float32)
    d2 = (qn - 2.0 * cross) + pn
    bmask = bq_ref[:] != bs
    d2 = jnp.where(bmask, 1e10, d2)

    iota = jax.lax.broadcasted_iota(jnp.int32, d2.shape, 1)
    S = None
    wsum = None
    for j in range(k):
        m = jnp.min(d2, axis=1, keepdims=True)
        am = jnp.min(jnp.where(d2 == m, iota, n_src), axis=1, keepdims=True)
        sel = iota == am
        if k == 1:
            S = sel.astype(jnp.float32)
        else:
            w = 1.0 / jnp.clip(m, 1e-16, None)
            contrib = jnp.where(sel, w, 0.0)
            S = contrib if S is None else S + contrib
            wsum = w if wsum is None else wsum + w
            if j + 1 < k:
                d2 = jnp.where(sel, 1e30, d2)

    out = jnp.dot(S, x, preferred_element_type=jnp.float32)
    if k > 1:
        out = out / wsum
    if affine:
        scale, shift = _coeffs(s_ref, ss_ref, g_ref, be_ref, n_prev)
        out = out * scale + shift
    o_ref[:] = out.astype(o_ref.dtype)


def _interp(x, pos_src_t, pos_q, bs_row, bq_col, k, stats=None, n_prev=None,
            bq=256, starts=None, win=None):
    n_src = pos_src_t.shape[1]
    n_q = pos_q.shape[0]
    c = x.shape[1]
    bq = min(bq, n_q)
    grid = (n_q // bq,)
    if win is None or win >= n_src:
        starts, win = None, None
    nsp = 1 if starts is not None else 0
    ix = lambda i: (i, 0)
    fix = lambda i: (0, 0)
    if nsp:
        ix = lambda i, _s: (i, 0)
        fix = lambda i, _s: (0, 0)
    in_specs = [
        pl.BlockSpec((bq, 1), ix),
        pl.BlockSpec((bq, 3), ix),
        pl.BlockSpec((1, n_src), fix),
        pl.BlockSpec((3, n_src), fix),
        pl.BlockSpec((n_src, c), fix),
    ]
    args = [bq_col, pos_q, bs_row, pos_src_t, x]
    if stats is not None:
        cs = pl.BlockSpec((1, c), fix)
        in_specs += [cs, cs, cs, cs]
        s, ss, g, be = stats
        args += [s, ss, g.reshape(1, c), be.reshape(1, c)]
    body = functools.partial(_interp_body, k=k, n_src=n_src,
                             affine=stats is not None,
                             n_prev=float(n_prev) if n_prev else 1.0,
                             win=win)
    out_spec = pl.BlockSpec((bq, c), ix)
    out_shape = jax.ShapeDtypeStruct((n_q, c), jnp.bfloat16)
    if nsp:
        gs = pltpu.PrefetchScalarGridSpec(
            num_scalar_prefetch=1, grid=grid,
            in_specs=in_specs, out_specs=out_spec)
        return pl.pallas_call(body, grid_spec=gs,
                              out_shape=out_shape)(starts, *args)
    return pl.pallas_call(
        body,
        grid=grid,
        in_specs=in_specs,
        out_specs=out_spec,
        out_shape=out_shape,
    )(*args)


def _interp_lin_body(st_ref, bq_ref, q_ref, bs_ref, pt_ref, x_ref,
                     s_ref, ss_ref, g_ref, be_ref, w_ref, b_ref, xs_ref,
                     h_ref, so_ref, sso_ref, *, win, n_prev, kx):
    st = pl.multiple_of(st_ref[pl.program_id(0)], 128)
    pt = pt_ref[:, pl.ds(st, win)]
    bs = bs_ref[:, pl.ds(st, win)]
    x = x_ref[pl.ds(st, win), :]
    q = q_ref[:]
    q0, q1, q2 = q[:, 0:1], q[:, 1:2], q[:, 2:3]
    p0, p1, p2 = pt[0:1, :], pt[1:2, :], pt[2:3, :]
    qn = q0 * q0 + q1 * q1 + q2 * q2
    pn = p0 * p0 + p1 * p1 + p2 * p2
    cross = jnp.dot(q.astype(jnp.bfloat16), pt.astype(jnp.bfloat16),
                    preferred_element_type=jnp.float32)
    d2 = (qn - 2.0 * cross) + pn
    d2 = jnp.where(bq_ref[:] != bs, 1e10, d2)

    iota = jax.lax.broadcasted_iota(jnp.int32, d2.shape, 1)
    S = None
    wsum = None
    for j in range(3):
        m = jnp.min(d2, axis=1, keepdims=True)
        am = jnp.min(jnp.where(d2 == m, iota, win), axis=1, keepdims=True)
        sel = iota == am
        w = 1.0 / jnp.clip(m, 1e-16, None)
        contrib = jnp.where(sel, w, 0.0)
        S = contrib if S is None else S + contrib
        wsum = w if wsum is None else wsum + w
        if j < 2:
            d2 = jnp.where(sel, 1e30, d2)

    xi = jnp.dot(S, x, preferred_element_type=jnp.float32) / wsum
    scale, shift = _coeffs(s_ref, ss_ref, g_ref, be_ref, n_prev)
    xi = (xi * scale + shift).astype(jnp.bfloat16)
    h = jnp.dot(xi, w_ref[0:kx, :].astype(jnp.bfloat16),
                preferred_element_type=jnp.float32)
    h = h + jnp.dot(xs_ref[:], w_ref[kx:, :],
                    preferred_element_type=jnp.float32)
    h = jnp.maximum(h + b_ref[:], 0.0)
    h_ref[:] = h
    csum = jnp.sum(h, axis=0, keepdims=True)
    csq = jnp.sum(h * h, axis=0, keepdims=True)

    @pl.when(pl.program_id(0) == 0)
    def _init():
        so_ref[:] = csum
        sso_ref[:] = csq

    @pl.when(pl.program_id(0) > 0)
    def _acc():
        so_ref[:] += csum
        sso_ref[:] += csq


def _interp_lin(x, pos_src_t, pos_q, bs_row, bq_col, stats, n_prev,
                w, b_row, skip, starts, win, bq=256):
    n_src = pos_src_t.shape[1]
    n_q = pos_q.shape[0]
    c = x.shape[1]
    fout = w.shape[1]
    ks = skip.shape[1]
    grid = (n_q // bq,)
    ix = lambda i, _s: (i, 0)
    fix = lambda i, _s: (0, 0)
    cs = pl.BlockSpec((1, c), fix)
    s, ss, g, be = stats
    gs = pltpu.PrefetchScalarGridSpec(
        num_scalar_prefetch=1, grid=grid,
        in_specs=[
            pl.BlockSpec((bq, 1), ix),
            pl.BlockSpec((bq, 3), ix),
            pl.BlockSpec((1, n_src), fix),
            pl.BlockSpec((3, n_src), fix),
            pl.BlockSpec((n_src, c), fix),
            cs, cs, cs, cs,
            pl.BlockSpec(w.shape, fix),
            pl.BlockSpec((1, fout), fix),
            pl.BlockSpec((bq, ks), ix),
        ],
        out_specs=[
            pl.BlockSpec((bq, fout), ix),
            pl.BlockSpec((1, fout), fix),
            pl.BlockSpec((1, fout), fix),
        ])
    return pl.pallas_call(
        functools.partial(_interp_lin_body, win=win, n_prev=float(n_prev),
                          kx=c),
        grid_spec=gs,
        out_shape=[
            jax.ShapeDtypeStruct((n_q, fout), jnp.float32),
            jax.ShapeDtypeStruct((1, fout), jnp.float32),
            jax.ShapeDtypeStruct((1, fout), jnp.float32),
        ],
    )(starts, bq_col, pos_q, bs_row, pos_src_t, x,
      s, ss, g.reshape(1, c), be.reshape(1, c), w, b_row, skip)


def _linear_body(x_ref, w_ref, b_ref, *rest, affine, skip, kx, n_prev):
    rest = list(rest)
    xs_ref = rest.pop(0) if skip else None
    if affine:
        s_ref, ss_ref, g_ref, be_ref = rest[:4]
        rest = rest[4:]
    h_ref, so_ref, sso_ref = rest
    x = x_ref[:]
    if affine:
        scale, shift = _coeffs(s_ref, ss_ref, g_ref, be_ref, n_prev)
        x = x * scale + shift
    w_head = w_ref[0:kx, :]
    if x.dtype == jnp.bfloat16:
        w_head = w_head.astype(jnp.bfloat16)
    h = jnp.dot(x, w_head, preferred_element_type=jnp.float32)
    if skip:
        h = h + jnp.dot(xs_ref[:], w_ref[kx:, :],
                        preferred_element_type=jnp.float32)
    h = h + b_ref[:]
    h = jnp.maximum(h, 0.0)
    h_ref[:] = h
    csum = jnp.sum(h, axis=0, keepdims=True)
    csq = jnp.sum(h * h, axis=0, keepdims=True)

    @pl.when(pl.program_id(0) == 0)
    def _init():
        so_ref[:] = csum
        sso_ref[:] = csq

    @pl.when(pl.program_id(0) > 0)
    def _acc():
        so_ref[:] += csum
        sso_ref[:] += csq


def _linear(x, w, b_row, skip=None, stats=None, n_prev=None, br=512):
    n, kx = x.shape
    fout = w.shape[1]
    br = min(br, n)
    grid = (n // br,)
    in_specs = [
        pl.BlockSpec((br, kx), lambda i: (i, 0)),
        pl.BlockSpec(w.shape, lambda i: (0, 0)),
        pl.BlockSpec((1, fout), lambda i: (0, 0)),
    ]
    args = [x, w, b_row]
    if skip is not None:
        ks = skip.shape[1]
        in_specs.append(pl.BlockSpec((br, ks), lambda i: (i, 0)))
        args.append(skip)
    if stats is not None:
        cs = pl.BlockSpec((1, kx), lambda i: (0, 0))
        in_specs += [cs, cs, cs, cs]
        s, ss, g, be = stats
        args += [s, ss, g.reshape(1, kx), be.reshape(1, kx)]
    return pl.pallas_call(
        functools.partial(_linear_body, affine=stats is not None,
                          skip=skip is not None, kx=kx,
                          n_prev=float(n_prev) if n_prev else 1.0),
        grid=grid,
        in_specs=in_specs,
        out_specs=[
            pl.BlockSpec((br, fout), lambda i: (i, 0)),
            pl.BlockSpec((1, fout), lambda i: (0, 0)),
            pl.BlockSpec((1, fout), lambda i: (0, 0)),
        ],
        out_shape=[
            jax.ShapeDtypeStruct((n, fout), jnp.float32),
            jax.ShapeDtypeStruct((1, fout), jnp.float32),
            jax.ShapeDtypeStruct((1, fout), jnp.float32),
        ],
    )(*args)


def _bn_apply_body(h_ref, s_ref, ss_ref, g_ref, be_ref, o_ref, *, n_prev):
    scale, shift = _coeffs(s_ref, ss_ref, g_ref, be_ref, n_prev)
    o_ref[:] = h_ref[:] * scale + shift


def _bn_apply(h, stats, n_prev, br=2048):
    n, f = h.shape
    s, ss, g, be = stats
    br = min(br, n)
    grid = (n // br,)
    cs = pl.BlockSpec((1, f), lambda i: (0, 0))
    return pl.pallas_call(
        functools.partial(_bn_apply_body, n_prev=float(n_prev)),
        grid=grid,
        in_specs=[pl.BlockSpec((br, f), lambda i: (i, 0)), cs, cs, cs, cs],
        out_specs=pl.BlockSpec((br, f), lambda i: (i, 0)),
        out_shape=jax.ShapeDtypeStruct((n, f), jnp.float32),
    )(h, s, ss, g.reshape(1, f), be.reshape(1, f))


def kernel(x0, x1, x2, x3, p0, p1, p2, p3, b0, b1, b2, b3,
           l3w1, l3b1, l3g1, l3be1, l3w2, l3b2, l3g2, l3be2,
           l2w1, l2b1, l2g1, l2be1, l2w2, l2b2, l2g2, l2be2,
           l1w1, l1b1, l1g1, l1be1, l1w2, l1b2, l1g2, l1be2):
    n0, n1, n2 = x0.shape[0], x1.shape[0], x2.shape[0]
    b0c = b0.reshape(-1, 1)
    b1c = b1.reshape(-1, 1)
    b2c = b2.reshape(-1, 1)
    b1r = b1.reshape(1, -1)
    b2r = b2.reshape(1, -1)
    b3r = b3.reshape(1, -1)

    h = _interp(x3, p3.T, p2, b3r, b2c, k=1)
    h, s, ss = _linear(h, l3w1, l3b1.reshape(1, -1), skip=x2)
    h, s, ss = _linear(h, l3w2, l3b2.reshape(1, -1),
                       stats=(s, ss, l3g1, l3be1), n_prev=n2)

    h = _interp(h, p2.T, p1, b2r, b1c, k=3,
                stats=(s, ss, l3g2, l3be2), n_prev=n2)
    h, s, ss = _linear(h, l2w1, l2b1.reshape(1, -1), skip=x1)
    h, s, ss = _linear(h, l2w2, l2b2.reshape(1, -1),
                       stats=(s, ss, l2g1, l2be1), n_prev=n1)

    win = 2560
    bq1 = 512
    nb = n0 // bq1
    fq = jnp.arange(nb) * bq1
    seg_start = jnp.searchsorted(b1, b0[fq], side="left").astype(jnp.int32)
    starts = jnp.minimum((seg_start // 128) * 128,
                         jnp.int32(x1.shape[0] - win))
    starts = jnp.maximum(starts, 0)
    h, s, ss = _interp_lin(h, p1.T, p0, b1r, b0c,
                           stats=(s, ss, l2g2, l2be2), n_prev=n1,
                           w=l1w1, b_row=l1b1.reshape(1, -1), skip=x0,
                           starts=starts, win=win, bq=bq1)
    h, s, ss = _linear(h, l1w2, l1b2.reshape(1, -1),
                       stats=(s, ss, l1g1, l1be1), n_prev=n0)

    return _bn_apply(h, (s, ss, l1g2, l1be2), n0)
